# baseline (speedup 1.0000x reference)
import functools

import jax
import jax.numpy as jnp
from jax import lax
from jax.experimental import pallas as pl
from jax.experimental.pallas import tpu as pltpu
from jax.experimental.pallas import tpu_sc as plsc

_NB = 16384
_NT = 50
_NF = 64
_V = 1000000
_VMAIN = 999936
_ACH = 31232
_ALC = 128
_QPW = 200
_H0 = 96
_H1 = 104


def _body(idx_flat, wt, tail_in, ot, wr, idx_raw, idx_g, a_in, a_tr,
          rows_v, b_out, a_tail, bsem, ais0, ais1, aos0, aos1, isem,
          gs0, gs1, os0, os1):
  cid = lax.axis_index("c")
  sid = lax.axis_index("s")
  w = sid * 2 + cid
  aises = (ais0, ais1)
  aoses = (aos0, aos1)
  gses = (gs0, gs1)
  oses = (os0, os1)
  iot = lax.iota(jnp.int32, 16)

  abase = w * _ACH
  nsub = jnp.where(w == 31, (_ACH + 512) // _ALC, _ACH // _ALC)

  def a_read(i, sl):
    off = pl.multiple_of(abase + i * _ALC, _ALC)
    return pltpu.make_async_copy(
        wt.at[:, pl.ds(off, _ALC)], a_in.at[sl], aises[sl])

  def a_write(i, sl):
    off = pl.multiple_of((abase + i * _ALC) // 2, _ALC // 2)
    return pltpu.make_async_copy(
        a_tr.at[sl], wr.at[pl.ds(off, _ALC // 2)], aoses[sl])

  a_read(0, 0).start()

  @pl.loop(0, nsub, step=2)
  def _aloop(i0):
    for sl in range(2):
      i = i0 + sl

      @pl.when(i + 1 < nsub)
      def _():
        a_read(i + 1, 1 - sl).start()

      a_read(i, sl).wait()

      @pl.when(i >= 2)
      def _():
        a_write(i - 2, sl).wait()

      @pl.loop(0, _ALC // 2)
      def _atr(u):
        for g in range(8):
          rloc = 2 * u + (1 if g >= 4 else 0)
          vals = plsc.load_gather(
              a_in.at[sl], [16 * (g % 4) + iot, iot * 0 + rloc])
          a_tr[sl, u, pl.ds(16 * g, 16)] = vals
      a_write(i, sl).start()

  a_write(nsub - 2, 0).wait()
  a_write(nsub - 1, 1).wait()

  @pl.when(w == 31)
  def _():
    tc = pltpu.make_async_copy(tail_in, a_tail, ais0)
    tc.start()
    tc.wait()
    for u in range(32):
      for g in range(8):
        a_tr[0, u, pl.ds(16 * g, 16)] = (
            a_tail[2 * u + (1 if g >= 4 else 0), pl.ds(16 * (g % 4), 16)])
    tw = pltpu.make_async_copy(
        a_tr.at[0, pl.ds(0, 32)], wr.at[pl.ds(_VMAIN // 2, 32)], aos0)
    tw.start()
    tw.wait()

  plsc.subcore_barrier()

  @pl.when(sid == 0)
  def _():
    pltpu.semaphore_signal(bsem, 1, core_index=1 - cid)
    pltpu.semaphore_wait(bsem, 1)

  plsc.subcore_barrier()

  qbase = w * _QPW

  def b_write(q, sl):
    t = q // 128
    bq = q % 128
    boff = pl.multiple_of(bq * 128, 128)
    return pltpu.make_async_copy(
        b_out.at[sl], ot.at[t, pl.ds(0, _NF), pl.ds(boff, 128)], oses[sl])

  for hh in range(2):
    nch = _H0 if hh == 0 else _H1
    hoff = 0 if hh == 0 else _H0 * 128

    idescs = [
        pltpu.make_async_copy(
            idx_flat.at[pl.ds(
                pl.multiple_of(w * _QPW * 128 + hoff + k * 1024, 1024),
                1024)],
            idx_raw.at[pl.ds(k * 1024, 1024)], isem)
        for k in range(nch * 128 // 1024)
    ]
    for d in idescs:
      d.start()
    for d in idescs:
      d.wait()

    @pl.loop(0, nch * 8)
    def _shift(k):
      v = idx_raw[pl.ds(k * 16, 16)]
      idx_g[pl.ds(k * 16, 16)] = lax.shift_right_logical(v, 1)

    def b_gather(lch, sl):
      ioff = pl.multiple_of(lch * 128, 128)
      return pltpu.make_async_copy(
          wr.at[idx_g.at[pl.ds(ioff, 128)]], rows_v.at[sl], gses[sl])

    @pl.loop(0, nch, step=2)
    def _bloop(ch0):
      for sl in range(2):
        lch = ch0 + sl

        @pl.when(ch0 >= 2)
        def _():
          b_write(qbase + hoff // 128 + lch - 2, sl).wait()

        b_gather(lch, sl).start()

      for sl in range(2):
        lch = ch0 + sl
        b_gather(lch, sl).wait()

        @pl.loop(0, 8)
        def _btr(jg):
          parv = lax.bitwise_and(
              idx_raw[pl.ds(pl.multiple_of(lch * 128 + 16 * jg, 16), 16)],
              1) * 64
          rowv = iot + 16 * jg
          for c in range(_NF):
            vals = plsc.load_gather(rows_v.at[sl], [rowv, parv + c])
            b_out[sl, c, pl.ds(16 * jg, 16)] = vals
        b_write(qbase + hoff // 128 + lch, sl).start()

    b_write(qbase + hoff // 128 + nch - 2, 0).wait()
    b_write(qbase + hoff // 128 + nch - 1, 1).wait()


@jax.jit
def _emb(idx_flat, wt, tail_in):
  mesh = plsc.VectorSubcoreMesh(
      core_axis_name="c", subcore_axis_name="s", num_cores=2, num_subcores=16)
  f = functools.partial(
      pl.kernel,
      mesh=mesh,
      out_type=(
          jax.ShapeDtypeStruct((_NT, _NF, _NB), jnp.float32),
          jax.ShapeDtypeStruct((_V // 2, 128), jnp.float32),
      ),
      scratch_types=[
          pltpu.VMEM((_H1 * 128,), jnp.int32),
          pltpu.VMEM((_H1 * 128,), jnp.int32),
          pltpu.VMEM((2, _NF, _ALC), jnp.float32),
          pltpu.VMEM((2, _ALC // 2, 128), jnp.float32),
          pltpu.VMEM((2, 128, 128), jnp.float32),
          pltpu.VMEM((2, _NF, 128), jnp.float32),
          pltpu.VMEM((64, 64), jnp.float32),
          pltpu.SemaphoreType.REGULAR,
          pltpu.SemaphoreType.DMA,
          pltpu.SemaphoreType.DMA,
          pltpu.SemaphoreType.DMA,
          pltpu.SemaphoreType.DMA,
          pltpu.SemaphoreType.DMA,
          pltpu.SemaphoreType.DMA,
          pltpu.SemaphoreType.DMA,
          pltpu.SemaphoreType.DMA,
          pltpu.SemaphoreType.DMA,
      ],
      compiler_params=pltpu.CompilerParams(
          use_tc_tiling_on_sc=True, needs_layout_passes=False),
  )(_body)
  return f(idx_flat, wt, tail_in)


def kernel(token_ids, weight):
  idx_flat = token_ids.T.astype(jnp.int32).reshape(_NT * _NB)
  wt = weight.T
  tail_in = weight[_VMAIN:, :]
  ot, _ = _emb(idx_flat, wt, tail_in)
  return jnp.transpose(ot, (2, 0, 1))


# speedup vs baseline: 2.3371x; 2.3371x over previous
import functools

import jax
import jax.numpy as jnp
from jax import lax
from jax.experimental import pallas as pl
from jax.experimental.pallas import tpu as pltpu
from jax.experimental.pallas import tpu_sc as plsc

_NUM_TOKENS = 16384
_SEQ = 50
_DIM = 64
_B = _NUM_TOKENS * _SEQ
_NC = 2
_NS = 16
_NW = _NC * _NS
_BPW = _B // _NW
_NBUF = 4
_CHUNK = 400
_G = 80
_GPC = _CHUNK // _G
_NCHUNKS = _BPW // _CHUNK


def _emb_body(ids_hbm, w_hbm, out_hbm, idx_v, rows_v, isems, gsems, osems):
  wid = lax.axis_index("s") * _NC + lax.axis_index("c")
  base = wid * _BPW

  def idx_desc(i, b):
    off = base + i * _CHUNK
    return pltpu.make_async_copy(
        ids_hbm.at[pl.ds(off, _CHUNK)], idx_v.at[b], isems[b])

  def gather_descs(b):
    return [
        pltpu.make_async_copy(
            w_hbm.at[idx_v.at[b, pl.ds(g * _G, _G)]],
            rows_v.at[b, pl.ds(g * _G, _G)],
            gsems[b])
        for g in range(_GPC)
    ]

  def out_desc(i, b):
    off = base + i * _CHUNK
    return pltpu.make_async_copy(
        rows_v.at[b], out_hbm.at[pl.ds(off, _CHUNK)], osems[b])

  idx_desc(0, 0).start()

  @pl.loop(0, _NCHUNKS, step=_NBUF)
  def _outer(i0):
    for b in range(_NBUF):
      i = i0 + b

      @pl.when(i + 1 < _NCHUNKS)
      def _():
        idx_desc(i + 1, (b + 1) % _NBUF).start()

      idx_desc(i, b).wait()

      @pl.when(i >= _NBUF)
      def _():
        out_desc(i - _NBUF, b).wait()

      for d in gather_descs(b):
        d.start()

      pb = (b + _NBUF - 1) % _NBUF

      @pl.when(i >= 1)
      def _():
        for d in gather_descs(pb):
          d.wait()
        out_desc(i - 1, pb).start()

  last = _NBUF - 1
  for d in gather_descs(last):
    d.wait()
  out_desc(_NCHUNKS - 1, last).start()
  for k in range(_NBUF):
    out_desc(_NCHUNKS - _NBUF + k, k).wait()


@jax.jit
def _emb(ids_flat, weight):
  mesh = plsc.VectorSubcoreMesh(
      core_axis_name="c", subcore_axis_name="s",
      num_cores=_NC, num_subcores=_NS,
  )
  f = functools.partial(
      pl.kernel,
      mesh=mesh,
      out_type=jax.ShapeDtypeStruct((_B, _DIM), jnp.float32),
      scratch_types=[
          pltpu.VMEM((_NBUF, _CHUNK), jnp.int32),
          pltpu.VMEM((_NBUF, _CHUNK, _DIM), jnp.float32),
          [pltpu.SemaphoreType.DMA] * _NBUF,
          [pltpu.SemaphoreType.DMA] * _NBUF,
          [pltpu.SemaphoreType.DMA] * _NBUF,
      ],
      compiler_params=pltpu.CompilerParams(use_tc_tiling_on_sc=False),
  )(_emb_body)
  return f(ids_flat, weight)


def kernel(token_ids, weight):
  ids_flat = token_ids.reshape(_B).astype(jnp.int32)
  out = _emb(ids_flat, weight)
  return out.reshape(_NUM_TOKENS, _SEQ, _DIM)
